# baseline (speedup 1.0000x reference)
import functools

import jax
import jax.numpy as jnp
from jax import lax
from jax.experimental import pallas as pl
from jax.experimental.pallas import tpu as pltpu
from jax.experimental.pallas import tpu_sc as plsc

N_RULES = 10000
K = 32
D = 128
N_SOURCE = 10000
NW = 32
CHUNK = 4

N_PAD = 10240
RULES_PER_W = N_PAD // NW
CHUNKS_PER_W = RULES_PER_W // CHUNK


def _sc_body(table_hbm, w_hbm, idx_hbm, out_hbm, w_v, idx_v, rows_v, out_v, sem):
    cid = lax.axis_index("c")
    sid = lax.axis_index("s")
    wid = sid * 2 + cid
    pltpu.sync_copy(w_hbm, w_v)
    rule_base = wid * RULES_PER_W

    def chunk_body(ci, carry):
        row0 = rule_base + ci * CHUNK
        pltpu.sync_copy(idx_hbm.at[pl.ds(row0 * K, CHUNK * K)], idx_v)
        pltpu.async_copy(table_hbm.at[idx_v], rows_v, sem).wait()
        for r in range(CHUNK):
            for db in range(D // 16):
                sl = pl.ds(db * 16, 16)
                acc = w_v[0, sl] * rows_v[r * K, sl]
                for k in range(1, K):
                    acc = acc + w_v[k, sl] * rows_v[r * K + k, sl]
                a = jnp.abs(acc)
                e = jnp.exp(a + a)
                t = 1.0 - 2.0 / (e + 1.0)
                out_v[r, sl] = jnp.sign(acc) * t
        pltpu.sync_copy(out_v, out_hbm.at[pl.ds(row0, CHUNK)])
        return carry

    lax.fori_loop(0, CHUNKS_PER_W, chunk_body, 0)


def kernel(layer_values, weights, gather_indices):
    table = layer_values.reshape(N_SOURCE, D)
    idx = gather_indices.astype(jnp.int32)
    idx = jnp.pad(idx, (0, (N_PAD - N_RULES) * K))

    mesh = plsc.VectorSubcoreMesh(core_axis_name="c", subcore_axis_name="s")
    run = pl.kernel(
        _sc_body,
        out_type=jax.ShapeDtypeStruct((N_PAD, D), jnp.float32),
        mesh=mesh,
        scratch_types=[
            pltpu.VMEM((K, D), jnp.float32),
            pltpu.VMEM((CHUNK * K,), jnp.int32),
            pltpu.VMEM((CHUNK * K, D), jnp.float32),
            pltpu.VMEM((CHUNK, D), jnp.float32),
            pltpu.SemaphoreType.DMA,
        ],
    )
    out = run(table, weights, idx)
    return out[:N_RULES].reshape(N_RULES, D, 1)


# speedup vs baseline: 1.5688x; 1.5688x over previous
import jax
import jax.numpy as jnp
from jax import lax
from jax.experimental import pallas as pl
from jax.experimental.pallas import tpu as pltpu
from jax.experimental.pallas import tpu_sc as plsc

N_RULES = 10000
K = 32
D = 128
N_SOURCE = 10000
NW = 32
CHUNK = 4

N_PAD = 10240
RULES_PER_W = N_PAD // NW
CHUNKS_PER_W = RULES_PER_W // CHUNK


def _sc_body(table_hbm, w_hbm, idx_hbm, out_hbm,
             w_v, idx_v, rows_a, rows_b, out_v, sem_a, sem_b):
    cid = lax.axis_index("c")
    sid = lax.axis_index("s")
    wid = sid * 2 + cid
    rule_base = wid * RULES_PER_W
    pltpu.sync_copy(w_hbm, w_v)
    pltpu.sync_copy(idx_hbm.at[wid], idx_v)

    rows = (rows_a, rows_b)
    sems = (sem_a, sem_b)
    pltpu.async_copy(table_hbm.at[idx_v.at[0]], rows[0], sems[0])

    def outer(c2, carry):
        for b in range(2):
            ci = c2 * 2 + b
            pltpu.make_async_copy(table_hbm.at[idx_v.at[ci]],
                                  rows[b], sems[b]).wait()

            @pl.when(ci + 1 < CHUNKS_PER_W)
            def _():
                pltpu.async_copy(table_hbm.at[idx_v.at[ci + 1]],
                                 rows[1 - b], sems[1 - b])

            for r in range(CHUNK):
                for db in range(D // 16):
                    sl = pl.ds(db * 16, 16)
                    acc = w_v[0, sl] * rows[b][r * K, sl]
                    for k in range(1, K):
                        acc = acc + w_v[k, sl] * rows[b][r * K + k, sl]
                    a = jnp.abs(acc)
                    e = jnp.exp(a + a)
                    t = 1.0 - 2.0 / (e + 1.0)
                    out_v[ci * CHUNK + r, sl] = jnp.sign(acc) * t
        return carry

    lax.fori_loop(0, CHUNKS_PER_W // 2, outer, 0)
    pltpu.sync_copy(out_v, out_hbm.at[pl.ds(rule_base, RULES_PER_W)])


def kernel(layer_values, weights, gather_indices):
    table = layer_values.reshape(N_SOURCE, D)
    idx = gather_indices.astype(jnp.int32)
    idx = jnp.pad(idx, (0, (N_PAD - N_RULES) * K))
    idx = idx.reshape(NW, CHUNKS_PER_W, CHUNK * K)

    mesh = plsc.VectorSubcoreMesh(core_axis_name="c", subcore_axis_name="s")
    run = pl.kernel(
        _sc_body,
        out_type=jax.ShapeDtypeStruct((N_PAD, D), jnp.float32),
        mesh=mesh,
        scratch_types=[
            pltpu.VMEM((K, D), jnp.float32),
            pltpu.VMEM((CHUNKS_PER_W, CHUNK * K), jnp.int32),
            pltpu.VMEM((CHUNK * K, D), jnp.float32),
            pltpu.VMEM((CHUNK * K, D), jnp.float32),
            pltpu.VMEM((RULES_PER_W, D), jnp.float32),
            pltpu.SemaphoreType.DMA,
            pltpu.SemaphoreType.DMA,
        ],
    )
    out = run(table, weights, idx)
    return out[:N_RULES].reshape(N_RULES, D, 1)


# speedup vs baseline: 4.2144x; 2.6863x over previous
import jax
import jax.numpy as jnp
from jax import lax
from jax.experimental import pallas as pl
from jax.experimental.pallas import tpu as pltpu
from jax.experimental.pallas import tpu_sc as plsc

N_RULES = 10000
K = 32
D = 128
N_SOURCE = 10000
NW = 32
CHUNK = 4

N_PAD = 10240
RULES_PER_W = N_PAD // NW
CHUNKS_PER_W = RULES_PER_W // CHUNK
IDX_PER_W = RULES_PER_W * K


def _sc_body(table_hbm, w_hbm, idx_hbm, out_hbm,
             tab_s, w_v, ib_a, ib_b, rows_a, rows_b, ob_a, ob_b,
             sem_a, sem_b, isem_a, isem_b, osem_a, osem_b):
    cid = lax.axis_index("c")
    sid = lax.axis_index("s")
    wid = sid * 2 + cid
    rule_base = wid * RULES_PER_W
    idx_base = wid * IDX_PER_W

    @pl.when(sid == 0)
    def _():
        pltpu.sync_copy(table_hbm, tab_s)

    pltpu.sync_copy(w_hbm, w_v)
    plsc.subcore_barrier()

    rows = (rows_a, rows_b)
    sems = (sem_a, sem_b)
    ibufs = (ib_a, ib_b)
    isems = (isem_a, isem_b)
    obufs = (ob_a, ob_b)
    osems = (osem_a, osem_b)

    pltpu.sync_copy(idx_hbm.at[pl.ds(idx_base, CHUNK * K)], ibufs[0])
    pltpu.async_copy(tab_s.at[ibufs[0]], rows[0], sems[0])
    pltpu.async_copy(idx_hbm.at[pl.ds(idx_base + CHUNK * K, CHUNK * K)],
                     ibufs[1], isems[1])

    def outer(cg, carry):
        for b in range(2):
            ci = cg * 2 + b
            row0 = rule_base + ci * CHUNK
            pltpu.make_async_copy(tab_s.at[ibufs[b]], rows[b], sems[b]).wait()

            @pl.when(ci + 1 < CHUNKS_PER_W)
            def _():
                pltpu.make_async_copy(
                    idx_hbm.at[pl.ds(idx_base + (ci + 1) * CHUNK * K,
                                     CHUNK * K)],
                    ibufs[1 - b], isems[1 - b]).wait()
                pltpu.async_copy(tab_s.at[ibufs[1 - b]],
                                 rows[1 - b], sems[1 - b])

            @pl.when(ci + 2 < CHUNKS_PER_W)
            def _():
                pltpu.async_copy(
                    idx_hbm.at[pl.ds(idx_base + (ci + 2) * CHUNK * K,
                                     CHUNK * K)],
                    ibufs[b], isems[b])

            @pl.when(cg > 0)
            def _():
                pltpu.make_async_copy(obufs[b],
                                      out_hbm.at[pl.ds(row0, CHUNK)],
                                      osems[b]).wait()

            def db_body(db, c2, rows_b=rows[b], obuf_b=obufs[b]):
                sl = pl.ds(db * 16, 16)
                for r in range(CHUNK):
                    acc = w_v[0, sl] * rows_b[r * K, sl]
                    for k in range(1, K):
                        acc = acc + w_v[k, sl] * rows_b[r * K + k, sl]
                    a = jnp.abs(acc)
                    e = jnp.exp(a + a)
                    t = 1.0 - 2.0 / (e + 1.0)
                    obuf_b[r, sl] = jnp.sign(acc) * t
                return c2

            lax.fori_loop(0, D // 16, db_body, 0)
            pltpu.async_copy(obufs[b], out_hbm.at[pl.ds(row0, CHUNK)],
                             osems[b])
        return carry

    lax.fori_loop(0, CHUNKS_PER_W // 2, outer, 0)
    last0 = rule_base + (CHUNKS_PER_W - 2) * CHUNK
    last1 = rule_base + (CHUNKS_PER_W - 1) * CHUNK
    pltpu.make_async_copy(obufs[0], out_hbm.at[pl.ds(last0, CHUNK)],
                          osems[0]).wait()
    pltpu.make_async_copy(obufs[1], out_hbm.at[pl.ds(last1, CHUNK)],
                          osems[1]).wait()


def kernel(layer_values, weights, gather_indices):
    table = layer_values.reshape(N_SOURCE, D)
    idx = gather_indices.astype(jnp.int32)
    idx = jnp.pad(idx, (0, (N_PAD - N_RULES) * K))

    mesh = plsc.VectorSubcoreMesh(core_axis_name="c", subcore_axis_name="s")
    run = pl.kernel(
        _sc_body,
        out_type=jax.ShapeDtypeStruct((N_PAD, D), jnp.float32),
        mesh=mesh,
        scratch_types=[
            pltpu.VMEM_SHARED((N_SOURCE, D), jnp.float32),
            pltpu.VMEM((K, D), jnp.float32),
            pltpu.VMEM((CHUNK * K,), jnp.int32),
            pltpu.VMEM((CHUNK * K,), jnp.int32),
            pltpu.VMEM((CHUNK * K, D), jnp.float32),
            pltpu.VMEM((CHUNK * K, D), jnp.float32),
            pltpu.VMEM((CHUNK, D), jnp.float32),
            pltpu.VMEM((CHUNK, D), jnp.float32),
            pltpu.SemaphoreType.DMA,
            pltpu.SemaphoreType.DMA,
            pltpu.SemaphoreType.DMA,
            pltpu.SemaphoreType.DMA,
            pltpu.SemaphoreType.DMA,
            pltpu.SemaphoreType.DMA,
        ],
    )
    out = run(table, weights, idx)
    return out[:N_RULES].reshape(N_RULES, D, 1)


# speedup vs baseline: 6.3133x; 1.4980x over previous
import jax
import jax.numpy as jnp
from jax import lax
from jax.experimental import pallas as pl
from jax.experimental.pallas import tpu as pltpu
from jax.experimental.pallas import tpu_sc as plsc

N_RULES = 10000
K = 32
D = 128
N_SOURCE = 10000
NW = 32
CHUNK = 4

N_PAD = 10240
RULES_PER_W = N_PAD // NW
CHUNKS_PER_W = RULES_PER_W // CHUNK
IDX_PER_W = RULES_PER_W * K


def _sc_body(table_hbm, w_hbm, idx_hbm, out_hbm,
             tab_s, w_v, ib_a, ib_b, rows_a, rows_b, ob_a, ob_b,
             sem_a, sem_b, isem_a, isem_b, osem_a, osem_b):
    cid = lax.axis_index("c")
    sid = lax.axis_index("s")
    wid = sid * 2 + cid
    rule_base = wid * RULES_PER_W
    idx_base = wid * IDX_PER_W

    @pl.when(sid == 0)
    def _():
        pltpu.sync_copy(table_hbm, tab_s)

    pltpu.sync_copy(w_hbm, w_v)
    plsc.subcore_barrier()

    rows = (rows_a, rows_b)
    sems = (sem_a, sem_b)
    ibufs = (ib_a, ib_b)
    isems = (isem_a, isem_b)
    obufs = (ob_a, ob_b)
    osems = (osem_a, osem_b)

    pltpu.sync_copy(idx_hbm.at[pl.ds(idx_base, CHUNK * K)], ibufs[0])
    pltpu.async_copy(tab_s.at[ibufs[0]], rows[0], sems[0])
    pltpu.async_copy(idx_hbm.at[pl.ds(idx_base + CHUNK * K, CHUNK * K)],
                     ibufs[1], isems[1])

    def outer(cg, carry):
        for b in range(2):
            ci = cg * 2 + b
            row0 = rule_base + ci * CHUNK
            pltpu.make_async_copy(tab_s.at[ibufs[b]], rows[b], sems[b]).wait()

            @pl.when(ci + 1 < CHUNKS_PER_W)
            def _():
                pltpu.make_async_copy(
                    idx_hbm.at[pl.ds(idx_base + (ci + 1) * CHUNK * K,
                                     CHUNK * K)],
                    ibufs[1 - b], isems[1 - b]).wait()
                pltpu.async_copy(tab_s.at[ibufs[1 - b]],
                                 rows[1 - b], sems[1 - b])

            @pl.when(ci + 2 < CHUNKS_PER_W)
            def _():
                pltpu.async_copy(
                    idx_hbm.at[pl.ds(idx_base + (ci + 2) * CHUNK * K,
                                     CHUNK * K)],
                    ibufs[b], isems[b])

            @pl.when(cg > 0)
            def _():
                pltpu.make_async_copy(obufs[b],
                                      out_hbm.at[pl.ds(row0, CHUNK)],
                                      osems[b]).wait()

            def db_body(db, c2, rows_b=rows[b], obuf_b=obufs[b]):
                sl = pl.ds(db * 16, 16)
                wk = w_v[0, sl]
                accs = [wk * rows_b[r * K, sl] for r in range(CHUNK)]
                for k in range(1, K):
                    wk = w_v[k, sl]
                    accs = [accs[r] + wk * rows_b[r * K + k, sl]
                            for r in range(CHUNK)]
                for r in range(CHUNK):
                    a = jnp.abs(accs[r])
                    e = jnp.exp(a + a)
                    t = 1.0 - 2.0 / (e + 1.0)
                    obuf_b[r, sl] = jnp.sign(accs[r]) * t
                return c2

            lax.fori_loop(0, D // 16, db_body, 0)
            pltpu.async_copy(obufs[b], out_hbm.at[pl.ds(row0, CHUNK)],
                             osems[b])
        return carry

    lax.fori_loop(0, CHUNKS_PER_W // 2, outer, 0)
    last0 = rule_base + (CHUNKS_PER_W - 2) * CHUNK
    last1 = rule_base + (CHUNKS_PER_W - 1) * CHUNK
    pltpu.make_async_copy(obufs[0], out_hbm.at[pl.ds(last0, CHUNK)],
                          osems[0]).wait()
    pltpu.make_async_copy(obufs[1], out_hbm.at[pl.ds(last1, CHUNK)],
                          osems[1]).wait()


def kernel(layer_values, weights, gather_indices):
    table = layer_values.reshape(N_SOURCE, D)
    idx = gather_indices.astype(jnp.int32)
    idx = jnp.pad(idx, (0, (N_PAD - N_RULES) * K))

    mesh = plsc.VectorSubcoreMesh(core_axis_name="c", subcore_axis_name="s")
    run = pl.kernel(
        _sc_body,
        out_type=jax.ShapeDtypeStruct((N_PAD, D), jnp.float32),
        mesh=mesh,
        scratch_types=[
            pltpu.VMEM_SHARED((N_SOURCE, D), jnp.float32),
            pltpu.VMEM((K, D), jnp.float32),
            pltpu.VMEM((CHUNK * K,), jnp.int32),
            pltpu.VMEM((CHUNK * K,), jnp.int32),
            pltpu.VMEM((CHUNK * K, D), jnp.float32),
            pltpu.VMEM((CHUNK * K, D), jnp.float32),
            pltpu.VMEM((CHUNK, D), jnp.float32),
            pltpu.VMEM((CHUNK, D), jnp.float32),
            pltpu.SemaphoreType.DMA,
            pltpu.SemaphoreType.DMA,
            pltpu.SemaphoreType.DMA,
            pltpu.SemaphoreType.DMA,
            pltpu.SemaphoreType.DMA,
            pltpu.SemaphoreType.DMA,
        ],
    )
    out = run(table, weights, idx)
    return out[:N_RULES].reshape(N_RULES, D, 1)
